# baseline (speedup 1.0000x reference)
import functools

import jax
import jax.numpy as jnp
from jax.experimental import pallas as pl
from jax.experimental.pallas import tpu as pltpu

NCLS = 21
B = 32
HB = 16
P = 8732
PPAD = 8960
BP = 1280
G = PPAD // BP
INT_MAX = 2147483647
NEG_INF = float("-inf")


def _pass1(yp_ref, yt_ref, step, nb):
    x = yp_ref[:, 0:24, :]
    cio = jax.lax.broadcasted_iota(jnp.int32, (nb, 24, BP), 1)
    xm = jnp.where(cio < NCLS, x, NEG_INF)
    m = jnp.max(xm, axis=1, keepdims=True)
    lse = jnp.log(jnp.sum(jnp.exp(xm - m), axis=1, keepdims=True)) + m
    lse2 = lse[:, 0, :]
    lbl = yt_ref[:, 4, :]
    lbl_i = lbl.astype(jnp.int32)
    sel = jnp.sum(jnp.where(cio == lbl_i[:, None, :], x, 0.0), axis=1)
    ce = lse2 - sel
    loss = lse2 - x[:, 0, :]
    gidx = step * BP + jax.lax.broadcasted_iota(jnp.int32, (nb, BP), 1)
    valid = gidx < P
    posm = (lbl > 0.0) & valid

    bits = jax.lax.bitcast_convert_type(loss, jnp.int32)
    key = jnp.where(posm, -1, bits)
    key = jnp.where(valid, key, -2)
    ce = jnp.where(valid, ce, 0.0)

    d = yp_ref[:, NCLS:NCLS + 4, :] - yt_ref[:, 0:4, :]
    ad = jnp.abs(d)
    hub = jnp.where(ad <= 1.0, 0.5 * d * d, ad - 0.5)
    hub = jnp.where(posm[:, None, :], hub, 0.0)
    return key, ce, posm.astype(jnp.float32), hub


def _body_a(yp_ref, yt_ref, key_o, ce_o, pos_o, hub_o):
    step = pl.program_id(0)

    @pl.when(step == 0)
    def _init():
        pos_o[...] = jnp.zeros_like(pos_o)
        hub_o[...] = jnp.zeros_like(hub_o)

    key, ce, posf, hub = _pass1(yp_ref, yt_ref, step, HB)
    key_o[...] = key
    ce_o[...] = ce
    pos_o[...] += posf
    hub_o[...] += hub


def _body_b(yp_ref, yt_ref, ak_ref, ac_ref, ap_ref, ah_ref, out_ref,
            key_s, ce_s, pos_s, hub_s):
    step = pl.program_id(0)

    @pl.when(step == 0)
    def _init():
        key_s[0:HB, :] = ak_ref[...]
        ce_s[0:HB, :] = ac_ref[...]
        pos_s[0:HB, :] = ap_ref[...]
        hub_s[0:HB, :, :] = ah_ref[...]
        pos_s[HB:B, :] = jnp.zeros((HB, BP), jnp.float32)
        hub_s[HB:B, :, :] = jnp.zeros((HB, 4, BP), jnp.float32)

    key, ce, posf, hub = _pass1(yp_ref, yt_ref, step, HB)
    key_s[HB:B, pl.ds(step * BP, BP)] = key
    ce_s[HB:B, pl.ds(step * BP, BP)] = ce
    pos_s[HB:B, :] += posf
    hub_s[HB:B, :, :] += hub

    @pl.when(step == G - 1)
    def _finish():
        npos_row = jnp.sum(pos_s[...], axis=1, keepdims=True)
        np_tot = jnp.sum(npos_row)
        k_eff = jnp.minimum(npos_row * 3.0, float(P))
        keys = key_s[...]

        def vbody(_, carry):
            lo, hi = carry
            mid = (lo >> 1) + (hi >> 1) + (lo & hi & 1)
            cnt = jnp.sum((keys >= mid).astype(jnp.float32), axis=1,
                          keepdims=True)
            ok = cnt >= k_eff
            return jnp.where(ok, mid, lo), jnp.where(ok, hi, mid)

        lo0 = jnp.full((B, 1), -2, jnp.int32)
        hi0 = jnp.full((B, 1), INT_MAX, jnp.int32)
        t, _ = jax.lax.fori_loop(0, 32, vbody, (lo0, hi0))
        cgt = jnp.sum((keys > t).astype(jnp.float32), axis=1, keepdims=True)
        r = k_eff - cgt
        tie = keys == t
        iot = jax.lax.broadcasted_iota(jnp.int32, (B, PPAD), 1)

        def ibody(i, s):
            cand = s + (8192 >> i)
            cnt2 = jnp.sum(jnp.where(tie & (iot < cand), 1.0, 0.0), axis=1,
                           keepdims=True)
            return jnp.where(cnt2 < r, cand, s)

        s = jax.lax.fori_loop(0, 14, ibody, jnp.zeros((B, 1), jnp.int32))
        mask = (keys > t) | (tie & (iot <= s)) | (keys == -1)
        cls = jnp.sum(jnp.where(mask, ce_s[...], 0.0))
        hub_tot = jnp.sum(hub_s[...])
        out_ref[0, 0] = hub_tot / (4.0 * np_tot * np_tot) + cls / np_tot


@functools.partial(jax.jit, static_argnames=("interpret",))
def kernel(y_true, y_pred, interpret=False):
    ypa_t = jnp.transpose(y_pred[:HB], (0, 2, 1))
    yta_t = jnp.transpose(y_true[:HB], (0, 2, 1))
    ypb_t = jnp.transpose(y_pred[HB:], (0, 2, 1))
    ytb_t = jnp.transpose(y_true[HB:], (0, 2, 1))

    ak, ac, ap, ah = pl.pallas_call(
        _body_a,
        grid=(G,),
        in_specs=[
            pl.BlockSpec((HB, 25, BP), lambda i: (0, 0, i)),
            pl.BlockSpec((HB, 5, BP), lambda i: (0, 0, i)),
        ],
        out_specs=[
            pl.BlockSpec((HB, BP), lambda i: (0, i)),
            pl.BlockSpec((HB, BP), lambda i: (0, i)),
            pl.BlockSpec((HB, BP), lambda i: (0, 0)),
            pl.BlockSpec((HB, 4, BP), lambda i: (0, 0, 0)),
        ],
        out_shape=[
            jax.ShapeDtypeStruct((HB, PPAD), jnp.int32),
            jax.ShapeDtypeStruct((HB, PPAD), jnp.float32),
            jax.ShapeDtypeStruct((HB, BP), jnp.float32),
            jax.ShapeDtypeStruct((HB, 4, BP), jnp.float32),
        ],
        interpret=interpret,
    )(ypa_t, yta_t)

    out = pl.pallas_call(
        _body_b,
        grid=(G,),
        in_specs=[
            pl.BlockSpec((HB, 25, BP), lambda i: (0, 0, i)),
            pl.BlockSpec((HB, 5, BP), lambda i: (0, 0, i)),
            pl.BlockSpec((HB, PPAD), lambda i: (0, 0)),
            pl.BlockSpec((HB, PPAD), lambda i: (0, 0)),
            pl.BlockSpec((HB, BP), lambda i: (0, 0)),
            pl.BlockSpec((HB, 4, BP), lambda i: (0, 0, 0)),
        ],
        out_specs=pl.BlockSpec(memory_space=pltpu.SMEM),
        out_shape=jax.ShapeDtypeStruct((1, 1), jnp.float32),
        scratch_shapes=[
            pltpu.VMEM((B, PPAD), jnp.int32),
            pltpu.VMEM((B, PPAD), jnp.float32),
            pltpu.VMEM((B, BP), jnp.float32),
            pltpu.VMEM((B, 4, BP), jnp.float32),
        ],
        interpret=interpret,
    )(ypb_t, ytb_t, ak, ac, ap, ah)
    return out[0, 0]


# speedup vs baseline: 1.0029x; 1.0029x over previous
import functools

import jax
import jax.numpy as jnp
from jax.experimental import pallas as pl
from jax.experimental.pallas import tpu as pltpu

NCLS = 21
B = 32
HB = 16
P = 8732
PPAD = 8960
BP = 1792
G = PPAD // BP
INT_MAX = 2147483647
NEG_INF = float("-inf")


def _pass1(yp_ref, yt_ref, step, nb):
    x = yp_ref[:, 0:24, :]
    cio = jax.lax.broadcasted_iota(jnp.int32, (nb, 24, BP), 1)
    xm = jnp.where(cio < NCLS, x, NEG_INF)
    m = jnp.max(xm, axis=1, keepdims=True)
    lse = jnp.log(jnp.sum(jnp.exp(xm - m), axis=1, keepdims=True)) + m
    lse2 = lse[:, 0, :]
    lbl = yt_ref[:, 4, :]
    lbl_i = lbl.astype(jnp.int32)
    sel = jnp.sum(jnp.where(cio == lbl_i[:, None, :], x, 0.0), axis=1)
    ce = lse2 - sel
    loss = lse2 - x[:, 0, :]
    gidx = step * BP + jax.lax.broadcasted_iota(jnp.int32, (nb, BP), 1)
    valid = gidx < P
    posm = (lbl > 0.0) & valid

    bits = jax.lax.bitcast_convert_type(loss, jnp.int32)
    key = jnp.where(posm, -1, bits)
    key = jnp.where(valid, key, -2)
    ce = jnp.where(valid, ce, 0.0)

    d = yp_ref[:, NCLS:NCLS + 4, :] - yt_ref[:, 0:4, :]
    ad = jnp.abs(d)
    hub = jnp.where(ad <= 1.0, 0.5 * d * d, ad - 0.5)
    hub = jnp.where(posm[:, None, :], hub, 0.0)
    return key, ce, posm.astype(jnp.float32), hub


def _body_a(yp_ref, yt_ref, key_o, ce_o, pos_o, hub_o):
    step = pl.program_id(0)

    @pl.when(step == 0)
    def _init():
        pos_o[...] = jnp.zeros_like(pos_o)
        hub_o[...] = jnp.zeros_like(hub_o)

    key, ce, posf, hub = _pass1(yp_ref, yt_ref, step, HB)
    key_o[...] = key
    ce_o[...] = ce
    pos_o[...] += posf
    hub_o[...] += hub


def _body_b(yp_ref, yt_ref, ak_ref, ac_ref, ap_ref, ah_ref, out_ref,
            key_s, ce_s, pos_s, hub_s):
    step = pl.program_id(0)

    @pl.when(step == 0)
    def _init():
        key_s[0:HB, :] = ak_ref[...]
        ce_s[0:HB, :] = ac_ref[...]
        pos_s[0:HB, :] = ap_ref[...]
        hub_s[0:HB, :, :] = ah_ref[...]
        pos_s[HB:B, :] = jnp.zeros((HB, BP), jnp.float32)
        hub_s[HB:B, :, :] = jnp.zeros((HB, 4, BP), jnp.float32)

    key, ce, posf, hub = _pass1(yp_ref, yt_ref, step, HB)
    key_s[HB:B, pl.ds(step * BP, BP)] = key
    ce_s[HB:B, pl.ds(step * BP, BP)] = ce
    pos_s[HB:B, :] += posf
    hub_s[HB:B, :, :] += hub

    @pl.when(step == G - 1)
    def _finish():
        npos_row = jnp.sum(pos_s[...], axis=1, keepdims=True)
        np_tot = jnp.sum(npos_row)
        k_eff = jnp.minimum(npos_row * 3.0, float(P))
        keys = key_s[...]

        def vbody(_, carry):
            lo, hi = carry
            mid = (lo >> 1) + (hi >> 1) + (lo & hi & 1)
            cnt = jnp.sum((keys >= mid).astype(jnp.float32), axis=1,
                          keepdims=True)
            ok = cnt >= k_eff
            return jnp.where(ok, mid, lo), jnp.where(ok, hi, mid)

        lo0 = jnp.full((B, 1), -2, jnp.int32)
        hi0 = jnp.full((B, 1), INT_MAX, jnp.int32)
        t, _ = jax.lax.fori_loop(0, 32, vbody, (lo0, hi0))
        cgt = jnp.sum((keys > t).astype(jnp.float32), axis=1, keepdims=True)
        r = k_eff - cgt
        tie = keys == t
        iot = jax.lax.broadcasted_iota(jnp.int32, (B, PPAD), 1)

        def ibody(i, s):
            cand = s + (8192 >> i)
            cnt2 = jnp.sum(jnp.where(tie & (iot < cand), 1.0, 0.0), axis=1,
                           keepdims=True)
            return jnp.where(cnt2 < r, cand, s)

        s = jax.lax.fori_loop(0, 14, ibody, jnp.zeros((B, 1), jnp.int32))
        mask = (keys > t) | (tie & (iot <= s)) | (keys == -1)
        cls = jnp.sum(jnp.where(mask, ce_s[...], 0.0))
        hub_tot = jnp.sum(hub_s[...])
        out_ref[0, 0] = hub_tot / (4.0 * np_tot * np_tot) + cls / np_tot


@functools.partial(jax.jit, static_argnames=("interpret",))
def kernel(y_true, y_pred, interpret=False):
    ypa_t = jnp.transpose(y_pred[:HB], (0, 2, 1))
    yta_t = jnp.transpose(y_true[:HB], (0, 2, 1))
    ypb_t = jnp.transpose(y_pred[HB:], (0, 2, 1))
    ytb_t = jnp.transpose(y_true[HB:], (0, 2, 1))

    ak, ac, ap, ah = pl.pallas_call(
        _body_a,
        grid=(G,),
        in_specs=[
            pl.BlockSpec((HB, 25, BP), lambda i: (0, 0, i)),
            pl.BlockSpec((HB, 5, BP), lambda i: (0, 0, i)),
        ],
        out_specs=[
            pl.BlockSpec((HB, BP), lambda i: (0, i)),
            pl.BlockSpec((HB, BP), lambda i: (0, i)),
            pl.BlockSpec((HB, BP), lambda i: (0, 0)),
            pl.BlockSpec((HB, 4, BP), lambda i: (0, 0, 0)),
        ],
        out_shape=[
            jax.ShapeDtypeStruct((HB, PPAD), jnp.int32),
            jax.ShapeDtypeStruct((HB, PPAD), jnp.float32),
            jax.ShapeDtypeStruct((HB, BP), jnp.float32),
            jax.ShapeDtypeStruct((HB, 4, BP), jnp.float32),
        ],
        interpret=interpret,
    )(ypa_t, yta_t)

    out = pl.pallas_call(
        _body_b,
        grid=(G,),
        in_specs=[
            pl.BlockSpec((HB, 25, BP), lambda i: (0, 0, i)),
            pl.BlockSpec((HB, 5, BP), lambda i: (0, 0, i)),
            pl.BlockSpec((HB, PPAD), lambda i: (0, 0)),
            pl.BlockSpec((HB, PPAD), lambda i: (0, 0)),
            pl.BlockSpec((HB, BP), lambda i: (0, 0)),
            pl.BlockSpec((HB, 4, BP), lambda i: (0, 0, 0)),
        ],
        out_specs=pl.BlockSpec(memory_space=pltpu.SMEM),
        out_shape=jax.ShapeDtypeStruct((1, 1), jnp.float32),
        scratch_shapes=[
            pltpu.VMEM((B, PPAD), jnp.int32),
            pltpu.VMEM((B, PPAD), jnp.float32),
            pltpu.VMEM((B, BP), jnp.float32),
            pltpu.VMEM((B, 4, BP), jnp.float32),
        ],
        interpret=interpret,
    )(ypb_t, ytb_t, ak, ac, ap, ah)
    return out[0, 0]
